# baseline (device time: 15893 ns/iter reference)
import jax
import jax.numpy as jnp
from jax import lax
from jax.experimental import pallas as pl
from jax.experimental.pallas import tpu as pltpu

N_DEV = 4
B = 2
S = 256
HQ = 4
DH = 64
BLK = 64
D_MODEL = 512

BF = jnp.bfloat16
F32 = jnp.float32
I8 = jnp.int8
QSCALE = 32.0


def kernel(x, Wq, K_ext, V_ext, Wo):
    wq16 = Wq.astype(BF)
    wo16 = Wo.astype(BF)
    kt8 = jnp.transpose(
        jnp.clip(jnp.round(K_ext * QSCALE), -127, 127).astype(I8),
        (0, 2, 3, 1))
    vt8 = jnp.transpose(
        jnp.clip(jnp.round(V_ext * QSCALE), -127, 127).astype(I8),
        (0, 2, 3, 1))

    def body(x_ref, wq_ref, kt8_ref, vt8_ref, wo_ref,
             out_ref, k_all, v_all, sk_sems, sv_sems, rk_sems, rv_sems):
        my = lax.axis_index("i")

        barrier = pltpu.get_barrier_semaphore()
        for t in range(N_DEV):
            @pl.when(my != t)
            def _():
                pl.semaphore_signal(
                    barrier, inc=1,
                    device_id=(t,), device_id_type=pl.DeviceIdType.MESH,
                )
        pl.semaphore_wait(barrier, N_DEV - 1)

        for s in range(N_DEV):
            for t in reversed(range(s + 1, N_DEV)):
                @pl.when(my == s)
                def _(s=s, t=t):
                    for b in range(B):
                        pltpu.make_async_remote_copy(
                            src_ref=kt8_ref.at[b], dst_ref=k_all.at[s, b],
                            send_sem=sk_sems.at[t, b],
                            recv_sem=rk_sems.at[s, b],
                            device_id=(t,),
                            device_id_type=pl.DeviceIdType.MESH,
                        ).start()
                    for b in range(B):
                        pltpu.make_async_remote_copy(
                            src_ref=vt8_ref.at[b], dst_ref=v_all.at[s, b],
                            send_sem=sv_sems.at[t, b],
                            recv_sem=rv_sems.at[s, b],
                            device_id=(t,),
                            device_id_type=pl.DeviceIdType.MESH,
                        ).start()

        q16 = [
            (jnp.dot(x_ref[b].astype(BF), wq_ref[...],
                     preferred_element_type=F32)
             * (0.125 / QSCALE)).astype(BF)
            for b in range(B)
        ]

        q_blk = lax.broadcasted_iota(jnp.int32, (S, S), 0) // BLK
        k_blk = lax.broadcasted_iota(jnp.int32, (S, S), 1) // BLK
        own_mask = k_blk <= q_blk

        l_sum = []
        acc = []
        for b in range(B):
            for h in range(HQ):
                qh = q16[b][:, h * DH:(h + 1) * DH]
                sc = jnp.dot(qh, kt8_ref[b, h].astype(BF),
                             preferred_element_type=F32)
                e = jnp.exp(jnp.where(own_mask, sc, -1e9))
                l_sum.append(jnp.sum(e, axis=1, keepdims=True))
                acc.append(lax.dot_general(
                    e.astype(BF), vt8_ref[b, h].astype(BF),
                    (((1,), (1,)), ((), ())),
                    preferred_element_type=F32))

        for s in (2, 0, 1):
            vis = my > s
            for b in range(B):
                @pl.when(my > s)
                def _(s=s, b=b):
                    pltpu.make_async_remote_copy(
                        src_ref=kt8_ref.at[b], dst_ref=k_all.at[s, b],
                        send_sem=sk_sems.at[s, b], recv_sem=rk_sems.at[s, b],
                        device_id=(s,), device_id_type=pl.DeviceIdType.MESH,
                    ).wait_recv()

                es = []
                for h in range(HQ):
                    qh = q16[b][:, h * DH:(h + 1) * DH]
                    sc = jnp.dot(qh, k_all[s, b, h].astype(BF),
                                 preferred_element_type=F32)
                    e = jnp.exp(jnp.where(vis, sc, -1e9))
                    i = b * HQ + h
                    l_sum[i] = l_sum[i] + jnp.sum(e, axis=1, keepdims=True)
                    es.append(e.astype(BF))

                @pl.when(my > s)
                def _(s=s, b=b):
                    pltpu.make_async_remote_copy(
                        src_ref=vt8_ref.at[b], dst_ref=v_all.at[s, b],
                        send_sem=sv_sems.at[s, b], recv_sem=rv_sems.at[s, b],
                        device_id=(s,), device_id_type=pl.DeviceIdType.MESH,
                    ).wait_recv()

                for h in range(HQ):
                    pv = lax.dot_general(
                        es[h], v_all[s, b, h].astype(BF),
                        (((1,), (1,)), ((), ())),
                        preferred_element_type=F32)
                    i = b * HQ + h
                    acc[i] = acc[i] + jnp.where(vis, pv, 0.0)

        for b in range(B):
            ctx = jnp.concatenate(
                [(acc[b * HQ + h] * (1.0 / QSCALE) / l_sum[b * HQ + h])
                 .astype(BF) for h in range(HQ)], axis=1)
            out_ref[b] = jnp.dot(ctx, wo_ref[...],
                                 preferred_element_type=F32)

        for s in range(N_DEV):
            for t in range(s + 1, N_DEV):
                @pl.when(my == s)
                def _(s=s, t=t):
                    for b in range(B):
                        pltpu.make_async_remote_copy(
                            src_ref=kt8_ref.at[b], dst_ref=k_all.at[s, b],
                            send_sem=sk_sems.at[t, b],
                            recv_sem=rk_sems.at[s, b],
                            device_id=(t,),
                            device_id_type=pl.DeviceIdType.MESH,
                        ).wait_send()
                        pltpu.make_async_remote_copy(
                            src_ref=vt8_ref.at[b], dst_ref=v_all.at[s, b],
                            send_sem=sv_sems.at[t, b],
                            recv_sem=rv_sems.at[s, b],
                            device_id=(t,),
                            device_id_type=pl.DeviceIdType.MESH,
                        ).wait_send()

    return pl.pallas_call(
        body,
        out_shape=jax.ShapeDtypeStruct((B, S, D_MODEL), F32),
        in_specs=[pl.BlockSpec(memory_space=pltpu.VMEM)] * 5,
        out_specs=pl.BlockSpec(memory_space=pltpu.VMEM),
        scratch_shapes=[
            pltpu.VMEM((N_DEV, B, HQ, DH, S), I8),
            pltpu.VMEM((N_DEV, B, HQ, DH, S), I8),
            pltpu.SemaphoreType.DMA((N_DEV, B)),
            pltpu.SemaphoreType.DMA((N_DEV, B)),
            pltpu.SemaphoreType.DMA((N_DEV, B)),
            pltpu.SemaphoreType.DMA((N_DEV, B)),
        ],
        compiler_params=pltpu.CompilerParams(collective_id=0),
    )(x, wq16, kt8, vt8, wo16)


# device time: 15088 ns/iter; 1.0534x vs baseline; 1.0534x over previous
import jax
import jax.numpy as jnp
from jax import lax
from jax.experimental import pallas as pl
from jax.experimental.pallas import tpu as pltpu

N_DEV = 4
B = 2
S = 256
HQ = 4
DH = 64
BLK = 64
D_MODEL = 512

BF = jnp.bfloat16
F32 = jnp.float32
I8 = jnp.int8
QSCALE = 32.0


def kernel(x, Wq, K_ext, V_ext, Wo):
    wq16 = Wq.astype(BF)
    wo16 = Wo.astype(BF)
    kt8 = jnp.transpose(
        jnp.clip(jnp.round(K_ext * QSCALE), -127, 127).astype(I8),
        (0, 2, 3, 1))
    vt8 = jnp.transpose(
        jnp.clip(jnp.round(V_ext * QSCALE), -127, 127).astype(I8),
        (0, 2, 3, 1))

    def body(x_ref, wq_ref, kt8_ref, vt8_ref, wo_ref,
             out_ref, k_all, v_all, sk_sems, sv_sems, rk_sems, rv_sems):
        my = lax.axis_index("i")

        barrier = pltpu.get_barrier_semaphore()
        for t in range(N_DEV):
            @pl.when(my != t)
            def _():
                pl.semaphore_signal(
                    barrier, inc=1,
                    device_id=(t,), device_id_type=pl.DeviceIdType.MESH,
                )
        pl.semaphore_wait(barrier, N_DEV - 1)

        for s in range(N_DEV):
            for t in reversed(range(s + 1, N_DEV)):
                @pl.when(my == s)
                def _(s=s, t=t):
                    pltpu.make_async_remote_copy(
                        src_ref=kt8_ref, dst_ref=k_all.at[s],
                        send_sem=sk_sems.at[t], recv_sem=rk_sems.at[s],
                        device_id=(t,), device_id_type=pl.DeviceIdType.MESH,
                    ).start()
                    pltpu.make_async_remote_copy(
                        src_ref=vt8_ref, dst_ref=v_all.at[s],
                        send_sem=sv_sems.at[t], recv_sem=rv_sems.at[s],
                        device_id=(t,), device_id_type=pl.DeviceIdType.MESH,
                    ).start()

        q16 = [
            (jnp.dot(x_ref[b].astype(BF), wq_ref[...],
                     preferred_element_type=F32)
             * (0.125 / QSCALE)).astype(BF)
            for b in range(B)
        ]

        q_blk = lax.broadcasted_iota(jnp.int32, (S, S), 0) // BLK
        k_blk = lax.broadcasted_iota(jnp.int32, (S, S), 1) // BLK
        own_mask = k_blk <= q_blk

        l_sum = []
        acc = []
        for b in range(B):
            for h in range(HQ):
                qh = q16[b][:, h * DH:(h + 1) * DH]
                sc = jnp.dot(qh, kt8_ref[b, h].astype(BF),
                             preferred_element_type=F32)
                e = jnp.exp(jnp.where(own_mask, sc, -1e9))
                l_sum.append(jnp.sum(e, axis=1, keepdims=True))
                acc.append(lax.dot_general(
                    e.astype(BF), vt8_ref[b, h].astype(BF),
                    (((1,), (1,)), ((), ())),
                    preferred_element_type=F32))

        for s in (2, 0, 1):
            @pl.when(my > s)
            def _(s=s):
                pltpu.make_async_remote_copy(
                    src_ref=kt8_ref, dst_ref=k_all.at[s],
                    send_sem=sk_sems.at[s], recv_sem=rk_sems.at[s],
                    device_id=(s,), device_id_type=pl.DeviceIdType.MESH,
                ).wait_recv()

            vis = my > s
            es = []
            i = 0
            for b in range(B):
                for h in range(HQ):
                    qh = q16[b][:, h * DH:(h + 1) * DH]
                    sc = jnp.dot(qh, k_all[s, b, h].astype(BF),
                                 preferred_element_type=F32)
                    e = jnp.exp(jnp.where(vis, sc, -1e9))
                    l_sum[i] = l_sum[i] + jnp.sum(e, axis=1, keepdims=True)
                    es.append(e.astype(BF))
                    i += 1

            @pl.when(my > s)
            def _(s=s):
                pltpu.make_async_remote_copy(
                    src_ref=vt8_ref, dst_ref=v_all.at[s],
                    send_sem=sv_sems.at[s], recv_sem=rv_sems.at[s],
                    device_id=(s,), device_id_type=pl.DeviceIdType.MESH,
                ).wait_recv()

            i = 0
            for b in range(B):
                for h in range(HQ):
                    pv = lax.dot_general(
                        es[i], v_all[s, b, h].astype(BF),
                        (((1,), (1,)), ((), ())),
                        preferred_element_type=F32)
                    acc[i] = acc[i] + jnp.where(vis, pv, 0.0)
                    i += 1

        for b in range(B):
            ctx = jnp.concatenate(
                [(acc[b * HQ + h] * (1.0 / QSCALE) / l_sum[b * HQ + h])
                 .astype(BF) for h in range(HQ)], axis=1)
            out_ref[b] = jnp.dot(ctx, wo_ref[...],
                                 preferred_element_type=F32)

        for s in range(N_DEV):
            for t in range(s + 1, N_DEV):
                @pl.when(my == s)
                def _(s=s, t=t):
                    pltpu.make_async_remote_copy(
                        src_ref=kt8_ref, dst_ref=k_all.at[s],
                        send_sem=sk_sems.at[t], recv_sem=rk_sems.at[s],
                        device_id=(t,), device_id_type=pl.DeviceIdType.MESH,
                    ).wait_send()
                    pltpu.make_async_remote_copy(
                        src_ref=vt8_ref, dst_ref=v_all.at[s],
                        send_sem=sv_sems.at[t], recv_sem=rv_sems.at[s],
                        device_id=(t,), device_id_type=pl.DeviceIdType.MESH,
                    ).wait_send()

    return pl.pallas_call(
        body,
        out_shape=jax.ShapeDtypeStruct((B, S, D_MODEL), F32),
        in_specs=[pl.BlockSpec(memory_space=pltpu.VMEM)] * 5,
        out_specs=pl.BlockSpec(memory_space=pltpu.VMEM),
        scratch_shapes=[
            pltpu.VMEM((N_DEV, B, HQ, DH, S), I8),
            pltpu.VMEM((N_DEV, B, HQ, DH, S), I8),
            pltpu.SemaphoreType.DMA((N_DEV,)),
            pltpu.SemaphoreType.DMA((N_DEV,)),
            pltpu.SemaphoreType.DMA((N_DEV,)),
            pltpu.SemaphoreType.DMA((N_DEV,)),
        ],
        compiler_params=pltpu.CompilerParams(collective_id=0),
    )(x, wq16, kt8, vt8, wo16)


# device time: 14785 ns/iter; 1.0749x vs baseline; 1.0205x over previous
import jax
import jax.numpy as jnp
from jax import lax
from jax.experimental import pallas as pl
from jax.experimental.pallas import tpu as pltpu

N_DEV = 4
B = 2
S = 256
HQ = 4
DH = 64
BLK = 64
D_MODEL = 512

BF = jnp.bfloat16
F32 = jnp.float32
I8 = jnp.int8
QSCALE = 32.0


def kernel(x, Wq, K_ext, V_ext, Wo):
    wq16 = Wq.astype(BF)
    wo16 = Wo.astype(BF)
    kt8 = jnp.transpose(
        jnp.clip(jnp.round(K_ext * QSCALE), -127, 127).astype(I8),
        (0, 2, 3, 1))
    vt8 = jnp.transpose(
        jnp.clip(jnp.round(V_ext * QSCALE), -127, 127).astype(I8),
        (0, 2, 3, 1))

    def body(x_ref, wq_ref, kt8_ref, vt8_ref, wo_ref,
             out_ref, k_all, v_all, sk_sems, sv_sems, rk_sems, rv_sems):
        my = lax.axis_index("i")

        barrier = pltpu.get_barrier_semaphore()
        for t in range(N_DEV):
            @pl.when(my != t)
            def _():
                pl.semaphore_signal(
                    barrier, inc=1,
                    device_id=(t,), device_id_type=pl.DeviceIdType.MESH,
                )
        pl.semaphore_wait(barrier, N_DEV - 1)

        for s in range(N_DEV):
            for t in reversed(range(s + 1, N_DEV)):
                @pl.when(my == s)
                def _(s=s, t=t):
                    pltpu.make_async_remote_copy(
                        src_ref=kt8_ref, dst_ref=k_all.at[s],
                        send_sem=sk_sems.at[t], recv_sem=rk_sems.at[s],
                        device_id=(t,), device_id_type=pl.DeviceIdType.MESH,
                    ).start()
                    pltpu.make_async_remote_copy(
                        src_ref=vt8_ref, dst_ref=v_all.at[s],
                        send_sem=sv_sems.at[t], recv_sem=rv_sems.at[s],
                        device_id=(t,), device_id_type=pl.DeviceIdType.MESH,
                    ).start()

        q16 = [
            (jnp.dot(x_ref[b].astype(BF), wq_ref[...],
                     preferred_element_type=F32)
             * (0.125 / QSCALE)).astype(BF)
            for b in range(B)
        ]

        l_sum = []
        acc = []
        for b in range(B):
            for h in range(HQ):
                qh = q16[b][:, h * DH:(h + 1) * DH]
                k_own = kt8_ref[b, h].astype(BF)
                v_own = vt8_ref[b, h].astype(BF)
                l_rows, a_rows = [], []
                for r in range(S // BLK):
                    w = (r + 1) * BLK
                    e_r = jnp.exp(jnp.dot(
                        qh[r * BLK:(r + 1) * BLK], k_own[:, :w],
                        preferred_element_type=F32))
                    l_rows.append(jnp.sum(e_r, axis=1, keepdims=True))
                    a_rows.append(lax.dot_general(
                        e_r.astype(BF), v_own[:, :w],
                        (((1,), (1,)), ((), ())),
                        preferred_element_type=F32))
                l_sum.append(jnp.concatenate(l_rows, axis=0))
                acc.append(jnp.concatenate(a_rows, axis=0))

        for s in (2, 0, 1):
            @pl.when(my > s)
            def _(s=s):
                pltpu.make_async_remote_copy(
                    src_ref=kt8_ref, dst_ref=k_all.at[s],
                    send_sem=sk_sems.at[s], recv_sem=rk_sems.at[s],
                    device_id=(s,), device_id_type=pl.DeviceIdType.MESH,
                ).wait_recv()

            vis = my > s
            es = []
            i = 0
            for b in range(B):
                for h in range(HQ):
                    qh = q16[b][:, h * DH:(h + 1) * DH]
                    sc = jnp.dot(qh, k_all[s, b, h].astype(BF),
                                 preferred_element_type=F32)
                    e = jnp.exp(jnp.where(vis, sc, -1e9))
                    l_sum[i] = l_sum[i] + jnp.sum(e, axis=1, keepdims=True)
                    es.append(e.astype(BF))
                    i += 1

            @pl.when(my > s)
            def _(s=s):
                pltpu.make_async_remote_copy(
                    src_ref=vt8_ref, dst_ref=v_all.at[s],
                    send_sem=sv_sems.at[s], recv_sem=rv_sems.at[s],
                    device_id=(s,), device_id_type=pl.DeviceIdType.MESH,
                ).wait_recv()

            i = 0
            for b in range(B):
                for h in range(HQ):
                    pv = lax.dot_general(
                        es[i], v_all[s, b, h].astype(BF),
                        (((1,), (1,)), ((), ())),
                        preferred_element_type=F32)
                    acc[i] = acc[i] + jnp.where(vis, pv, 0.0)
                    i += 1

        for b in range(B):
            ctx = jnp.concatenate(
                [(acc[b * HQ + h] * (1.0 / QSCALE) / l_sum[b * HQ + h])
                 .astype(BF) for h in range(HQ)], axis=1)
            out_ref[b] = jnp.dot(ctx, wo_ref[...],
                                 preferred_element_type=F32)

        for s in range(N_DEV):
            for t in range(s + 1, N_DEV):
                @pl.when(my == s)
                def _(s=s, t=t):
                    pltpu.make_async_remote_copy(
                        src_ref=kt8_ref, dst_ref=k_all.at[s],
                        send_sem=sk_sems.at[t], recv_sem=rk_sems.at[s],
                        device_id=(t,), device_id_type=pl.DeviceIdType.MESH,
                    ).wait_send()
                    pltpu.make_async_remote_copy(
                        src_ref=vt8_ref, dst_ref=v_all.at[s],
                        send_sem=sv_sems.at[t], recv_sem=rv_sems.at[s],
                        device_id=(t,), device_id_type=pl.DeviceIdType.MESH,
                    ).wait_send()

    return pl.pallas_call(
        body,
        out_shape=jax.ShapeDtypeStruct((B, S, D_MODEL), F32),
        in_specs=[pl.BlockSpec(memory_space=pltpu.VMEM)] * 5,
        out_specs=pl.BlockSpec(memory_space=pltpu.VMEM),
        scratch_shapes=[
            pltpu.VMEM((N_DEV, B, HQ, DH, S), I8),
            pltpu.VMEM((N_DEV, B, HQ, DH, S), I8),
            pltpu.SemaphoreType.DMA((N_DEV,)),
            pltpu.SemaphoreType.DMA((N_DEV,)),
            pltpu.SemaphoreType.DMA((N_DEV,)),
            pltpu.SemaphoreType.DMA((N_DEV,)),
        ],
        compiler_params=pltpu.CompilerParams(collective_id=0),
    )(x, wq16, kt8, vt8, wo16)


# device time: 14441 ns/iter; 1.1005x vs baseline; 1.0238x over previous
import jax
import jax.numpy as jnp
from jax import lax
from jax.experimental import pallas as pl
from jax.experimental.pallas import tpu as pltpu

N_DEV = 4
B = 2
S = 256
HQ = 4
DH = 64
BLK = 64
D_MODEL = 512

BF = jnp.bfloat16
F32 = jnp.float32
I8 = jnp.int8
QSCALE = 32.0


def kernel(x, Wq, K_ext, V_ext, Wo):
    wq16 = Wq.astype(BF)
    wo16 = Wo.astype(BF)
    kt8 = jnp.transpose(
        jnp.clip(jnp.round(K_ext * QSCALE), -127, 127).astype(I8),
        (0, 2, 3, 1))
    vt8 = jnp.transpose(
        jnp.clip(jnp.round(V_ext * QSCALE), -127, 127).astype(I8),
        (0, 2, 3, 1))

    def body(x_ref, wq_ref, kt8_ref, vt8_ref, wo_ref,
             out_ref, k_all, v_all, sk_sems, sv_sems, rk_sems, rv_sems):
        my = lax.axis_index("i")

        barrier = pltpu.get_barrier_semaphore()
        for t in range(N_DEV):
            @pl.when(my != t)
            def _():
                pl.semaphore_signal(
                    barrier, inc=1,
                    device_id=(t,), device_id_type=pl.DeviceIdType.MESH,
                )
        pl.semaphore_wait(barrier, N_DEV - 1)

        for s in range(N_DEV):
            for t in reversed(range(s + 1, N_DEV)):
                @pl.when(my == s)
                def _(s=s, t=t):
                    pltpu.make_async_remote_copy(
                        src_ref=kt8_ref, dst_ref=k_all.at[s],
                        send_sem=sk_sems.at[t], recv_sem=rk_sems.at[s],
                        device_id=(t,), device_id_type=pl.DeviceIdType.MESH,
                    ).start()
                    pltpu.make_async_remote_copy(
                        src_ref=vt8_ref, dst_ref=v_all.at[s],
                        send_sem=sv_sems.at[t], recv_sem=rv_sems.at[s],
                        device_id=(t,), device_id_type=pl.DeviceIdType.MESH,
                    ).start()

        q16 = [
            (jnp.dot(x_ref[b].astype(BF), wq_ref[...],
                     preferred_element_type=F32)
             * (0.125 / QSCALE)).astype(BF)
            for b in range(B)
        ]

        l_sum = []
        acc = []
        for b in range(B):
            for h in range(HQ):
                qh = q16[b][:, h * DH:(h + 1) * DH]
                k_own = kt8_ref[b, h].astype(BF)
                v_own = vt8_ref[b, h].astype(BF)
                l_rows, a_rows = [], []
                for r in range(S // BLK):
                    w = (r + 1) * BLK
                    e_r = jnp.exp(jnp.dot(
                        qh[r * BLK:(r + 1) * BLK], k_own[:, :w],
                        preferred_element_type=F32))
                    l_rows.append(jnp.sum(e_r, axis=1, keepdims=True))
                    a_rows.append(lax.dot_general(
                        e_r.astype(BF), v_own[:, :w],
                        (((1,), (1,)), ((), ())),
                        preferred_element_type=F32))
                l_sum.append(jnp.concatenate(l_rows, axis=0))
                acc.append(jnp.concatenate(a_rows, axis=0))

        for s in (2, 0, 1):
            @pl.when(my > s)
            def _(s=s):
                pltpu.make_async_remote_copy(
                    src_ref=kt8_ref, dst_ref=k_all.at[s],
                    send_sem=sk_sems.at[s], recv_sem=rk_sems.at[s],
                    device_id=(s,), device_id_type=pl.DeviceIdType.MESH,
                ).wait_recv()

            vis = my > s
            es = []
            i = 0
            for b in range(B):
                for h in range(HQ):
                    qh = q16[b][:, h * DH:(h + 1) * DH]
                    sc = jnp.dot(qh, k_all[s, b, h].astype(BF),
                                 preferred_element_type=F32)
                    e = jnp.exp(jnp.where(vis, sc, -1e9))
                    l_sum[i] = l_sum[i] + jnp.sum(e, axis=1, keepdims=True)
                    es.append(e.astype(BF))
                    i += 1

            @pl.when(my > s)
            def _(s=s):
                pltpu.make_async_remote_copy(
                    src_ref=vt8_ref, dst_ref=v_all.at[s],
                    send_sem=sv_sems.at[s], recv_sem=rv_sems.at[s],
                    device_id=(s,), device_id_type=pl.DeviceIdType.MESH,
                ).wait_recv()

            i = 0
            for b in range(B):
                for h in range(HQ):
                    pv = lax.dot_general(
                        es[i], v_all[s, b, h].astype(BF),
                        (((1,), (1,)), ((), ())),
                        preferred_element_type=F32)
                    acc[i] = acc[i] + jnp.where(vis, pv, 0.0)
                    i += 1

        for b in range(B):
            ctx = jnp.concatenate(
                [(acc[b * HQ + h]
                  * ((1.0 / QSCALE) / l_sum[b * HQ + h]))
                 .astype(BF) for h in range(HQ)], axis=1)
            out_ref[b] = jnp.dot(ctx, wo_ref[...],
                                 preferred_element_type=F32)

        for s in range(N_DEV):
            for t in range(s + 1, N_DEV):
                @pl.when(my == s)
                def _(s=s, t=t):
                    pltpu.make_async_remote_copy(
                        src_ref=kt8_ref, dst_ref=k_all.at[s],
                        send_sem=sk_sems.at[t], recv_sem=rk_sems.at[s],
                        device_id=(t,), device_id_type=pl.DeviceIdType.MESH,
                    ).wait_send()
                    pltpu.make_async_remote_copy(
                        src_ref=vt8_ref, dst_ref=v_all.at[s],
                        send_sem=sv_sems.at[t], recv_sem=rv_sems.at[s],
                        device_id=(t,), device_id_type=pl.DeviceIdType.MESH,
                    ).wait_send()

    return pl.pallas_call(
        body,
        out_shape=jax.ShapeDtypeStruct((B, S, D_MODEL), F32),
        in_specs=[pl.BlockSpec(memory_space=pltpu.VMEM)] * 5,
        out_specs=pl.BlockSpec(memory_space=pltpu.VMEM),
        scratch_shapes=[
            pltpu.VMEM((N_DEV, B, HQ, DH, S), I8),
            pltpu.VMEM((N_DEV, B, HQ, DH, S), I8),
            pltpu.SemaphoreType.DMA((N_DEV,)),
            pltpu.SemaphoreType.DMA((N_DEV,)),
            pltpu.SemaphoreType.DMA((N_DEV,)),
            pltpu.SemaphoreType.DMA((N_DEV,)),
        ],
        compiler_params=pltpu.CompilerParams(collective_id=0),
    )(x, wq16, kt8, vt8, wo16)
